# baseline (speedup 1.0000x reference)
import functools

import jax
import jax.numpy as jnp
from jax import lax
from jax.experimental import pallas as pl
from jax.experimental.pallas import tpu as pltpu
from jax.experimental.pallas import tpu_sc as plsc

THRESHOLD = 0.5
VARIANCES = (0.1, 0.2)
NEGPOS_RATIO = 3.0
LANES = 128
P_REAL = 8732
P_OUT = 8736
INF_BITS = 0x7F800001


def _tc_body(x_ref, tgt_ref, pri_ref, out_ref, bits_ref, stat_ref, *, P, C, T):
    tgt = tgt_ref[0]
    txmin = tgt[:, 0:1]
    tymin = tgt[:, 1:2]
    txmax = tgt[:, 2:3]
    tymax = tgt[:, 3:4]
    tlabel = tgt[:, 4:5]

    pcx = pri_ref[0:1, :]
    pcy = pri_ref[1:2, :]
    pw = pri_ref[2:3, :]
    ph = pri_ref[3:4, :]
    pxmin = pcx - pw / 2.0
    pymin = pcy - ph / 2.0
    pxmax = pcx + pw / 2.0
    pymax = pcy + ph / 2.0

    ix = jnp.clip(jnp.minimum(txmax, pxmax) - jnp.maximum(txmin, pxmin), 0.0, None)
    iy = jnp.clip(jnp.minimum(tymax, pymax) - jnp.maximum(tymin, pymin), 0.0, None)
    inter = ix * iy
    area_t = (txmax - txmin) * (tymax - tymin)
    area_p = (pxmax - pxmin) * (pymax - pymin)
    ov = inter / (area_t + area_p - inter)

    t_iota = jax.lax.broadcasted_iota(jnp.int32, (T, P), 0)
    p_iota = jax.lax.broadcasted_iota(jnp.int32, (T, P), 1)

    btv = jnp.max(ov, axis=0, keepdims=True)
    bti = jnp.min(jnp.where(ov == btv, t_iota, T), axis=0, keepdims=True)
    bpv = jnp.max(ov, axis=1, keepdims=True)
    bpi = jnp.min(jnp.where(ov == bpv, p_iota, P), axis=1, keepdims=True)

    m = bpi == p_iota
    ch_t = jnp.max(jnp.where(m, t_iota, -1), axis=0, keepdims=True)
    forced = ch_t >= 0
    bti = jnp.where(forced, ch_t, bti)
    btv = jnp.where(forced, 2.0, btv)

    oh = (t_iota == bti).astype(jnp.float32)
    matched = jax.lax.dot_general(tgt, oh, (((0,), (0,)), ((), ())),
                                  preferred_element_type=jnp.float32,
                                  precision=jax.lax.Precision.HIGHEST)
    mx0 = matched[0:1, :]
    my0 = matched[1:2, :]
    mx1 = matched[2:3, :]
    my1 = matched[3:4, :]
    mlab = matched[4:5, :]

    cls = jnp.where(btv < THRESHOLD, 0.0, mlab)
    cls_i = cls.astype(jnp.int32)
    pos = cls_i > 0

    gx = ((mx0 + mx1) / 2.0 - pcx) / (VARIANCES[0] * pw)
    gy = ((my0 + my1) / 2.0 - pcy) / (VARIANCES[0] * ph)
    gw = jnp.log((mx1 - mx0) / pw + 1e-8) / VARIANCES[1]
    gh = jnp.log((my1 - my0) / ph + 1e-8) / VARIANCES[1]

    loc = x_ref[0, C:C + 4]
    posf = pos.astype(jnp.float32)

    def sl1(d):
        ad = jnp.abs(d)
        return jnp.where(ad < 1.0, 0.5 * ad * ad, ad - 0.5)

    loss_l_b = jnp.sum((sl1(loc[0:1, :] - gx) + sl1(loc[1:2, :] - gy) +
                        sl1(loc[2:3, :] - gw) + sl1(loc[3:4, :] - gh)) * posf)

    conf = x_ref[0, 0:C]
    cmax = jnp.max(conf, axis=0, keepdims=True)
    sexp = jnp.sum(jnp.exp(conf - cmax), axis=0, keepdims=True)
    lse = cmax + jnp.log(sexp)
    c_iota = jax.lax.broadcasted_iota(jnp.int32, (C, P), 0)
    gathered = jnp.sum(jnp.where(c_iota == cls_i, conf, 0.0),
                       axis=0, keepdims=True)
    ce = lse - gathered

    npos_b = jnp.sum(posf)
    spce_b = jnp.sum(ce * posf)

    loss_c = jnp.where(pos, 0.0, ce)
    zpad = jnp.zeros((1, P_OUT - P), jnp.float32)
    row = jnp.concatenate([loss_c, zpad], axis=1)
    out_ref[...] = row.reshape(1, 1, P_OUT)
    bits_ref[...] = jax.lax.bitcast_convert_type(row, jnp.int32).reshape(1, 1, P_OUT)
    lane128 = jax.lax.broadcasted_iota(jnp.int32, (1, 128), 1)
    strow = (jnp.where(lane128 == 0, npos_b, 0.0)
             + jnp.where(lane128 == 1, loss_l_b, 0.0)
             + jnp.where(lane128 == 2, spce_b, 0.0))
    stat_ref[...] = strow.reshape(1, 1, 128)


def _vreduce(v, op):
    lane = lax.iota(jnp.int32, 16)
    dnums = jax.lax.GatherDimensionNumbers(
        offset_dims=(), collapsed_slice_dims=(0,), start_index_map=(0,))
    for sh in (8, 4, 2, 1):
        idx = jnp.bitwise_and(lane + sh, 15)
        perm = jax.lax.gather(v, idx[:, None], dnums, slice_sizes=(1,),
                              mode=jax.lax.GatherScatterMode.PROMISE_IN_BOUNDS)
        v = op(v, perm)
    return v[0]


def _sc_topk_body(lc_hbm, bits_hbm, st_hbm, out_hbm, row_v, bits_v, st_v, out_v):
    nc = 2
    wid = lax.axis_index("s") * nc + lax.axis_index("c")
    pltpu.sync_copy(lc_hbm.at[wid], row_v)
    pltpu.sync_copy(bits_hbm.at[wid], bits_v)
    pltpu.sync_copy(st_hbm.at[wid], st_v)

    lane = lax.iota(jnp.int32, 16)
    st = st_v[pl.ds(0, 16)]
    npos = st[0]
    lossl = st[1]
    spce = st[2]
    k_f = jnp.minimum(NEGPOS_RATIO * npos, float(P_REAL - 1))

    nchunks = P_OUT // 16

    def count_ge(mid):
        def chunk(j, cv):
            vb = bits_v[pl.ds(j * 16, 16)]
            return cv + jnp.where(vb >= mid, 1, 0)
        cv = lax.fori_loop(0, nchunks, chunk, jnp.zeros((16,), jnp.int32),
                           unroll=8)
        return _vreduce(cv, jnp.add)

    def bstep(_, carry):
        lo, hi = carry
        mid = lo + (hi - lo) // 2
        ge = count_ge(mid).astype(jnp.float32) >= k_f
        return jnp.where(ge, mid, lo), jnp.where(ge, hi, mid)

    lo, _ = lax.fori_loop(0, 31, bstep, (jnp.int32(0), jnp.int32(INF_BITS)))

    def tailsum(j, carry):
        sv, cv, mv = carry
        v = row_v[pl.ds(j * 16, 16)]
        gt = bits_v[pl.ds(j * 16, 16)] > lo
        return (sv + jnp.where(gt, v, 0.0), cv + jnp.where(gt, 1, 0),
                jnp.maximum(mv, jnp.where(gt, -1.0, v)))

    sv, cv, mv = lax.fori_loop(0, nchunks, tailsum,
                               (jnp.zeros((16,), jnp.float32),
                                jnp.zeros((16,), jnp.int32),
                                jnp.full((16,), -1.0, jnp.float32)), unroll=8)
    sum_gt = _vreduce(sv, jnp.add)
    cnt_gt = _vreduce(cv, jnp.add).astype(jnp.float32)
    tval = _vreduce(mv, jnp.maximum)
    s_top = sum_gt + jnp.where(k_f > cnt_gt, (k_f - cnt_gt) * tval, 0.0)

    out_v[...] = (jnp.where(lane == 0, s_top, 0.0)
                  + jnp.where(lane == 1, npos, 0.0)
                  + jnp.where(lane == 2, lossl, 0.0)
                  + jnp.where(lane == 3, spce, 0.0))
    pltpu.sync_copy(out_v, out_hbm.at[wid])


def kernel(loc_data, conf_data, targets, priors):
    B, P, C = conf_data.shape
    T = targets.shape[1]
    x_r = jnp.transpose(jnp.concatenate([conf_data, loc_data], axis=2),
                        (0, 2, 1))
    pri_r = priors.T
    lc, lc_bits, stats = pl.pallas_call(
        functools.partial(_tc_body, P=P, C=C, T=T),
        grid=(B,),
        in_specs=[
            pl.BlockSpec((1, C + 4, P), lambda b: (b, 0, 0)),
            pl.BlockSpec((1, T, 5), lambda b: (b, 0, 0)),
            pl.BlockSpec((4, P), lambda b: (0, 0)),
        ],
        out_specs=[
            pl.BlockSpec((1, 1, P_OUT), lambda b: (b, 0, 0)),
            pl.BlockSpec((1, 1, P_OUT), lambda b: (b, 0, 0)),
            pl.BlockSpec((1, 1, 128), lambda b: (b, 0, 0)),
        ],
        out_shape=[
            jax.ShapeDtypeStruct((B, 1, P_OUT), jnp.float32),
            jax.ShapeDtypeStruct((B, 1, P_OUT), jnp.int32),
            jax.ShapeDtypeStruct((B, 1, 128), jnp.float32),
        ],
    )(x_r, targets, pri_r)
    lc = lc.reshape(B, P_OUT)
    lc_bits = lc_bits.reshape(B, P_OUT)
    stats = stats.reshape(B, 128)

    mesh = plsc.VectorSubcoreMesh(core_axis_name="c", subcore_axis_name="s")
    partials = pl.kernel(
        _sc_topk_body,
        mesh=mesh,
        out_type=jax.ShapeDtypeStruct((B, 16), jnp.float32),
        scratch_types=[
            pltpu.VMEM((P_OUT,), jnp.float32),
            pltpu.VMEM((P_OUT,), jnp.int32),
            pltpu.VMEM((128,), jnp.float32),
            pltpu.VMEM((16,), jnp.float32),
        ],
    )(lc, lc_bits, stats)

    s_top = partials[:, 0]
    npos = partials[:, 1]
    lossl = partials[:, 2]
    spce = partials[:, 3]
    n_total = jnp.sum(npos)
    return (jnp.sum(lossl) + jnp.sum(spce) + jnp.sum(s_top)) / n_total


# speedup vs baseline: 1.1930x; 1.1930x over previous
import functools

import jax
import jax.numpy as jnp
from jax import lax
from jax.experimental import pallas as pl
from jax.experimental.pallas import tpu as pltpu
from jax.experimental.pallas import tpu_sc as plsc

THRESHOLD = 0.5
VARIANCES = (0.1, 0.2)
NEGPOS_RATIO = 3.0
LANES = 128
P_REAL = 8732
P_OUT = 8736
INF_BITS = 0x7F800001


def _tc_body(conf_ref, loc_ref, tgt_ref, pri_ref, out_ref, bits_ref, stat_ref, *, P, C, T):
    tgt = tgt_ref[0]
    txmin = tgt[:, 0:1]
    tymin = tgt[:, 1:2]
    txmax = tgt[:, 2:3]
    tymax = tgt[:, 3:4]
    tlabel = tgt[:, 4:5]

    pcx = pri_ref[0:1, :]
    pcy = pri_ref[1:2, :]
    pw = pri_ref[2:3, :]
    ph = pri_ref[3:4, :]
    pxmin = pcx - pw / 2.0
    pymin = pcy - ph / 2.0
    pxmax = pcx + pw / 2.0
    pymax = pcy + ph / 2.0

    ix = jnp.clip(jnp.minimum(txmax, pxmax) - jnp.maximum(txmin, pxmin), 0.0, None)
    iy = jnp.clip(jnp.minimum(tymax, pymax) - jnp.maximum(tymin, pymin), 0.0, None)
    inter = ix * iy
    area_t = (txmax - txmin) * (tymax - tymin)
    area_p = (pxmax - pxmin) * (pymax - pymin)
    ov = inter / (area_t + area_p - inter)

    t_iota = jax.lax.broadcasted_iota(jnp.int32, (T, P), 0)
    p_iota = jax.lax.broadcasted_iota(jnp.int32, (T, P), 1)

    btv = jnp.max(ov, axis=0, keepdims=True)
    bti = jnp.min(jnp.where(ov == btv, t_iota, T), axis=0, keepdims=True)
    bpv = jnp.max(ov, axis=1, keepdims=True)
    bpi = jnp.min(jnp.where(ov == bpv, p_iota, P), axis=1, keepdims=True)

    m = bpi == p_iota
    ch_t = jnp.max(jnp.where(m, t_iota, -1), axis=0, keepdims=True)
    forced = ch_t >= 0
    bti = jnp.where(forced, ch_t, bti)
    btv = jnp.where(forced, 2.0, btv)

    oh = (t_iota == bti).astype(jnp.float32)
    matched = jax.lax.dot_general(tgt, oh, (((0,), (0,)), ((), ())),
                                  preferred_element_type=jnp.float32,
                                  precision=jax.lax.Precision.HIGHEST)
    mx0 = matched[0:1, :]
    my0 = matched[1:2, :]
    mx1 = matched[2:3, :]
    my1 = matched[3:4, :]
    mlab = matched[4:5, :]

    cls = jnp.where(btv < THRESHOLD, 0.0, mlab)
    cls_i = cls.astype(jnp.int32)
    pos = cls_i > 0

    gx = ((mx0 + mx1) / 2.0 - pcx) / (VARIANCES[0] * pw)
    gy = ((my0 + my1) / 2.0 - pcy) / (VARIANCES[0] * ph)
    gw = jnp.log((mx1 - mx0) / pw + 1e-8) / VARIANCES[1]
    gh = jnp.log((my1 - my0) / ph + 1e-8) / VARIANCES[1]

    loc = loc_ref[0]
    posf = pos.astype(jnp.float32)

    def sl1(d):
        ad = jnp.abs(d)
        return jnp.where(ad < 1.0, 0.5 * ad * ad, ad - 0.5)

    loss_l_b = jnp.sum((sl1(loc[0:1, :] - gx) + sl1(loc[1:2, :] - gy) +
                        sl1(loc[2:3, :] - gw) + sl1(loc[3:4, :] - gh)) * posf)

    conf = conf_ref[0]
    cmax = jnp.max(conf, axis=0, keepdims=True)
    sexp = jnp.sum(jnp.exp(conf - cmax), axis=0, keepdims=True)
    lse = cmax + jnp.log(sexp)
    c_iota = jax.lax.broadcasted_iota(jnp.int32, (C, P), 0)
    gathered = jnp.sum(jnp.where(c_iota == cls_i, conf, 0.0),
                       axis=0, keepdims=True)
    ce = lse - gathered

    npos_b = jnp.sum(posf)
    spce_b = jnp.sum(ce * posf)

    loss_c = jnp.where(pos, 0.0, ce)
    zpad = jnp.zeros((1, P_OUT - P), jnp.float32)
    row = jnp.concatenate([loss_c, zpad], axis=1)
    out_ref[...] = row.reshape(1, 1, P_OUT)
    bits_ref[...] = jax.lax.bitcast_convert_type(row, jnp.int32).reshape(1, 1, P_OUT)
    lane128 = jax.lax.broadcasted_iota(jnp.int32, (1, 128), 1)
    strow = (jnp.where(lane128 == 0, npos_b, 0.0)
             + jnp.where(lane128 == 1, loss_l_b, 0.0)
             + jnp.where(lane128 == 2, spce_b, 0.0))
    stat_ref[...] = strow.reshape(1, 1, 128)


def _vreduce(v, op):
    lane = lax.iota(jnp.int32, 16)
    dnums = jax.lax.GatherDimensionNumbers(
        offset_dims=(), collapsed_slice_dims=(0,), start_index_map=(0,))
    for sh in (8, 4, 2, 1):
        idx = jnp.bitwise_and(lane + sh, 15)
        perm = jax.lax.gather(v, idx[:, None], dnums, slice_sizes=(1,),
                              mode=jax.lax.GatherScatterMode.PROMISE_IN_BOUNDS)
        v = op(v, perm)
    return v[0]


def _sc_topk_body(lc_hbm, bits_hbm, st_hbm, out_hbm, row_v, bits_v, st_v, out_v):
    nc = 2
    wid = lax.axis_index("s") * nc + lax.axis_index("c")
    pltpu.sync_copy(lc_hbm.at[wid], row_v)
    pltpu.sync_copy(bits_hbm.at[wid], bits_v)
    pltpu.sync_copy(st_hbm.at[wid], st_v)

    lane = lax.iota(jnp.int32, 16)
    st = st_v[pl.ds(0, 16)]
    npos = st[0]
    lossl = st[1]
    spce = st[2]
    k_f = jnp.minimum(NEGPOS_RATIO * npos, float(P_REAL - 1))

    nchunks = P_OUT // 16

    def count_ge(mid):
        def chunk(j, cv):
            vb = bits_v[pl.ds(j * 16, 16)]
            return cv + jnp.where(vb >= mid, 1, 0)
        cv = lax.fori_loop(0, nchunks, chunk, jnp.zeros((16,), jnp.int32),
                           unroll=16)
        return _vreduce(cv, jnp.add)

    def bstep(_, carry):
        lo, hi = carry
        mid = lo + (hi - lo) // 2
        ge = count_ge(mid).astype(jnp.float32) >= k_f
        return jnp.where(ge, mid, lo), jnp.where(ge, hi, mid)

    lo, _ = lax.fori_loop(0, 31, bstep, (jnp.int32(0), jnp.int32(INF_BITS)))

    def tailsum(j, carry):
        sv, cv, mv = carry
        v = row_v[pl.ds(j * 16, 16)]
        gt = bits_v[pl.ds(j * 16, 16)] > lo
        return (sv + jnp.where(gt, v, 0.0), cv + jnp.where(gt, 1, 0),
                jnp.maximum(mv, jnp.where(gt, -1.0, v)))

    sv, cv, mv = lax.fori_loop(0, nchunks, tailsum,
                               (jnp.zeros((16,), jnp.float32),
                                jnp.zeros((16,), jnp.int32),
                                jnp.full((16,), -1.0, jnp.float32)), unroll=8)
    sum_gt = _vreduce(sv, jnp.add)
    cnt_gt = _vreduce(cv, jnp.add).astype(jnp.float32)
    tval = _vreduce(mv, jnp.maximum)
    s_top = sum_gt + jnp.where(k_f > cnt_gt, (k_f - cnt_gt) * tval, 0.0)

    out_v[...] = (jnp.where(lane == 0, s_top, 0.0)
                  + jnp.where(lane == 1, npos, 0.0)
                  + jnp.where(lane == 2, lossl, 0.0)
                  + jnp.where(lane == 3, spce, 0.0))
    pltpu.sync_copy(out_v, out_hbm.at[wid])


def kernel(loc_data, conf_data, targets, priors):
    B, P, C = conf_data.shape
    T = targets.shape[1]
    conf_r = jnp.transpose(conf_data, (0, 2, 1))
    loc_r = jnp.transpose(loc_data, (0, 2, 1))
    pri_r = priors.T
    lc, lc_bits, stats = pl.pallas_call(
        functools.partial(_tc_body, P=P, C=C, T=T),
        grid=(B,),
        in_specs=[
            pl.BlockSpec((1, C, P), lambda b: (b, 0, 0)),
            pl.BlockSpec((1, 4, P), lambda b: (b, 0, 0)),
            pl.BlockSpec((1, T, 5), lambda b: (b, 0, 0)),
            pl.BlockSpec((4, P), lambda b: (0, 0)),
        ],
        out_specs=[
            pl.BlockSpec((1, 1, P_OUT), lambda b: (b, 0, 0)),
            pl.BlockSpec((1, 1, P_OUT), lambda b: (b, 0, 0)),
            pl.BlockSpec((1, 1, 128), lambda b: (b, 0, 0)),
        ],
        out_shape=[
            jax.ShapeDtypeStruct((B, 1, P_OUT), jnp.float32),
            jax.ShapeDtypeStruct((B, 1, P_OUT), jnp.int32),
            jax.ShapeDtypeStruct((B, 1, 128), jnp.float32),
        ],
    )(conf_r, loc_r, targets, pri_r)
    lc = lc.reshape(B, P_OUT)
    lc_bits = lc_bits.reshape(B, P_OUT)
    stats = stats.reshape(B, 128)

    mesh = plsc.VectorSubcoreMesh(core_axis_name="c", subcore_axis_name="s")
    partials = pl.kernel(
        _sc_topk_body,
        mesh=mesh,
        out_type=jax.ShapeDtypeStruct((B, 16), jnp.float32),
        scratch_types=[
            pltpu.VMEM((P_OUT,), jnp.float32),
            pltpu.VMEM((P_OUT,), jnp.int32),
            pltpu.VMEM((128,), jnp.float32),
            pltpu.VMEM((16,), jnp.float32),
        ],
    )(lc, lc_bits, stats)

    s_top = partials[:, 0]
    npos = partials[:, 1]
    lossl = partials[:, 2]
    spce = partials[:, 3]
    n_total = jnp.sum(npos)
    return (jnp.sum(lossl) + jnp.sum(spce) + jnp.sum(s_top)) / n_total


# speedup vs baseline: 1.1947x; 1.0014x over previous
import functools

import jax
import jax.numpy as jnp
from jax import lax
from jax.experimental import pallas as pl
from jax.experimental.pallas import tpu as pltpu
from jax.experimental.pallas import tpu_sc as plsc

THRESHOLD = 0.5
VARIANCES = (0.1, 0.2)
NEGPOS_RATIO = 3.0
P_REAL = 8732
P_OUT = 8736
INF_BITS = 0x7F800001


def _tc_body(conf_ref, loc_ref, tgt_ref, pri_ref, out_ref, bits_ref, stat_ref, *, P, C, T):
    tgt = tgt_ref[0]
    txmin = tgt[:, 0:1]
    tymin = tgt[:, 1:2]
    txmax = tgt[:, 2:3]
    tymax = tgt[:, 3:4]
    tlabel = tgt[:, 4:5]

    pcx = pri_ref[0:1, :]
    pcy = pri_ref[1:2, :]
    pw = pri_ref[2:3, :]
    ph = pri_ref[3:4, :]
    pxmin = pcx - pw / 2.0
    pymin = pcy - ph / 2.0
    pxmax = pcx + pw / 2.0
    pymax = pcy + ph / 2.0

    ix = jnp.clip(jnp.minimum(txmax, pxmax) - jnp.maximum(txmin, pxmin), 0.0, None)
    iy = jnp.clip(jnp.minimum(tymax, pymax) - jnp.maximum(tymin, pymin), 0.0, None)
    inter = ix * iy
    area_t = (txmax - txmin) * (tymax - tymin)
    area_p = (pxmax - pxmin) * (pymax - pymin)
    ov = inter / (area_t + area_p - inter)

    t_iota = jax.lax.broadcasted_iota(jnp.int32, (T, P), 0)
    p_iota = jax.lax.broadcasted_iota(jnp.int32, (T, P), 1)

    btv = jnp.max(ov, axis=0, keepdims=True)
    bti = jnp.min(jnp.where(ov == btv, t_iota, T), axis=0, keepdims=True)
    bpv = jnp.max(ov, axis=1, keepdims=True)
    bpi = jnp.min(jnp.where(ov == bpv, p_iota, P), axis=1, keepdims=True)

    m = bpi == p_iota
    ch_t = jnp.max(jnp.where(m, t_iota, -1), axis=0, keepdims=True)
    forced = ch_t >= 0
    bti = jnp.where(forced, ch_t, bti)
    btv = jnp.where(forced, 2.0, btv)

    oh = (t_iota == bti).astype(jnp.float32)
    matched = jax.lax.dot_general(tgt, oh, (((0,), (0,)), ((), ())),
                                  preferred_element_type=jnp.float32,
                                  precision=jax.lax.Precision.HIGHEST)
    mx0 = matched[0:1, :]
    my0 = matched[1:2, :]
    mx1 = matched[2:3, :]
    my1 = matched[3:4, :]
    mlab = matched[4:5, :]

    cls = jnp.where(btv < THRESHOLD, 0.0, mlab)
    cls_i = cls.astype(jnp.int32)
    pos = cls_i > 0

    gx = ((mx0 + mx1) / 2.0 - pcx) / (VARIANCES[0] * pw)
    gy = ((my0 + my1) / 2.0 - pcy) / (VARIANCES[0] * ph)
    gw = jnp.log((mx1 - mx0) / pw + 1e-8) / VARIANCES[1]
    gh = jnp.log((my1 - my0) / ph + 1e-8) / VARIANCES[1]

    loc = loc_ref[0]
    posf = pos.astype(jnp.float32)

    def sl1(d):
        ad = jnp.abs(d)
        return jnp.where(ad < 1.0, 0.5 * ad * ad, ad - 0.5)

    loss_l_b = jnp.sum((sl1(loc[0:1, :] - gx) + sl1(loc[1:2, :] - gy) +
                        sl1(loc[2:3, :] - gw) + sl1(loc[3:4, :] - gh)) * posf)

    conf = conf_ref[0]
    cmax = jnp.max(conf, axis=0, keepdims=True)
    sexp = jnp.sum(jnp.exp(conf - cmax), axis=0, keepdims=True)
    lse = cmax + jnp.log(sexp)
    c_iota = jax.lax.broadcasted_iota(jnp.int32, (C, P), 0)
    gathered = jnp.sum(jnp.where(c_iota == cls_i, conf, 0.0),
                       axis=0, keepdims=True)
    ce = lse - gathered

    npos_b = jnp.sum(posf)
    spce_b = jnp.sum(ce * posf)

    loss_c = jnp.where(pos, 0.0, ce)
    zpad = jnp.zeros((1, P_OUT - P), jnp.float32)
    row = jnp.concatenate([loss_c, zpad], axis=1)
    out_ref[...] = row.reshape(1, 1, P_OUT)
    bits_ref[...] = jax.lax.bitcast_convert_type(row, jnp.int32).reshape(1, 1, P_OUT)
    lane128 = jax.lax.broadcasted_iota(jnp.int32, (1, 128), 1)
    strow = (jnp.where(lane128 == 0, npos_b, 0.0)
             + jnp.where(lane128 == 1, loss_l_b, 0.0)
             + jnp.where(lane128 == 2, spce_b, 0.0))
    stat_ref[...] = strow.reshape(1, 1, 128)


def _vreduce(v, op):
    lane = lax.iota(jnp.int32, 16)
    dnums = jax.lax.GatherDimensionNumbers(
        offset_dims=(), collapsed_slice_dims=(0,), start_index_map=(0,))
    for sh in (8, 4, 2, 1):
        idx = jnp.bitwise_and(lane + sh, 15)
        perm = jax.lax.gather(v, idx[:, None], dnums, slice_sizes=(1,),
                              mode=jax.lax.GatherScatterMode.PROMISE_IN_BOUNDS)
        v = op(v, perm)
    return v[0]


def _sc_topk_body(lc_hbm, bits_hbm, st_hbm, out_hbm, row_v, bits_v, st_v, out_v):
    nc = 2
    wid = lax.axis_index("s") * nc + lax.axis_index("c")
    pltpu.sync_copy(lc_hbm.at[wid], row_v)
    pltpu.sync_copy(bits_hbm.at[wid], bits_v)
    pltpu.sync_copy(st_hbm.at[wid], st_v)

    lane = lax.iota(jnp.int32, 16)
    st = st_v[pl.ds(0, 16)]
    npos = st[0]
    lossl = st[1]
    spce = st[2]
    k_f = jnp.minimum(NEGPOS_RATIO * npos, float(P_REAL - 1))

    nchunks = P_OUT // 16

    def count_ge(mid):
        def chunk(j, cv):
            vb = bits_v[pl.ds(j * 16, 16)]
            return cv + jnp.where(vb >= mid, 1, 0)
        cv = lax.fori_loop(0, nchunks, chunk, jnp.zeros((16,), jnp.int32),
                           unroll=16)
        return _vreduce(cv, jnp.add)

    def bstep(_, carry):
        lo, hi = carry
        mid = lo + (hi - lo) // 2
        ge = count_ge(mid).astype(jnp.float32) >= k_f
        return jnp.where(ge, mid, lo), jnp.where(ge, hi, mid)

    lo, _ = lax.fori_loop(0, 31, bstep, (jnp.int32(0), jnp.int32(INF_BITS)))

    def tailsum(j, carry):
        sv, cv, mv = carry
        v = row_v[pl.ds(j * 16, 16)]
        gt = bits_v[pl.ds(j * 16, 16)] > lo
        return (sv + jnp.where(gt, v, 0.0), cv + jnp.where(gt, 1, 0),
                jnp.maximum(mv, jnp.where(gt, -1.0, v)))

    sv, cv, mv = lax.fori_loop(0, nchunks, tailsum,
                               (jnp.zeros((16,), jnp.float32),
                                jnp.zeros((16,), jnp.int32),
                                jnp.full((16,), -1.0, jnp.float32)), unroll=8)
    sum_gt = _vreduce(sv, jnp.add)
    cnt_gt = _vreduce(cv, jnp.add).astype(jnp.float32)
    tval = _vreduce(mv, jnp.maximum)
    s_top = sum_gt + jnp.where(k_f > cnt_gt, (k_f - cnt_gt) * tval, 0.0)

    out_v[...] = (jnp.where(lane == 0, s_top, 0.0)
                  + jnp.where(lane == 1, npos, 0.0)
                  + jnp.where(lane == 2, lossl, 0.0)
                  + jnp.where(lane == 3, spce, 0.0))
    pltpu.sync_copy(out_v, out_hbm.at[wid])


def kernel(loc_data, conf_data, targets, priors):
    B, P, C = conf_data.shape
    T = targets.shape[1]
    conf_r = jnp.transpose(conf_data, (0, 2, 1))
    loc_r = jnp.transpose(loc_data, (0, 2, 1))
    pri_r = priors.T
    lc, lc_bits, stats = pl.pallas_call(
        functools.partial(_tc_body, P=P, C=C, T=T),
        grid=(B,),
        in_specs=[
            pl.BlockSpec((1, C, P), lambda b: (b, 0, 0)),
            pl.BlockSpec((1, 4, P), lambda b: (b, 0, 0)),
            pl.BlockSpec((1, T, 5), lambda b: (b, 0, 0)),
            pl.BlockSpec((4, P), lambda b: (0, 0)),
        ],
        out_specs=[
            pl.BlockSpec((1, 1, P_OUT), lambda b: (b, 0, 0)),
            pl.BlockSpec((1, 1, P_OUT), lambda b: (b, 0, 0)),
            pl.BlockSpec((1, 1, 128), lambda b: (b, 0, 0)),
        ],
        out_shape=[
            jax.ShapeDtypeStruct((B, 1, P_OUT), jnp.float32),
            jax.ShapeDtypeStruct((B, 1, P_OUT), jnp.int32),
            jax.ShapeDtypeStruct((B, 1, 128), jnp.float32),
        ],
    )(conf_r, loc_r, targets, pri_r)
    lc = lc.reshape(B, P_OUT)
    lc_bits = lc_bits.reshape(B, P_OUT)
    stats = stats.reshape(B, 128)

    mesh = plsc.VectorSubcoreMesh(core_axis_name="c", subcore_axis_name="s")
    partials = pl.kernel(
        _sc_topk_body,
        mesh=mesh,
        out_type=jax.ShapeDtypeStruct((B, 16), jnp.float32),
        scratch_types=[
            pltpu.VMEM((P_OUT,), jnp.float32),
            pltpu.VMEM((P_OUT,), jnp.int32),
            pltpu.VMEM((128,), jnp.float32),
            pltpu.VMEM((16,), jnp.float32),
        ],
    )(lc, lc_bits, stats)

    s_top = partials[:, 0]
    npos = partials[:, 1]
    lossl = partials[:, 2]
    spce = partials[:, 3]
    n_total = jnp.sum(npos)
    return (jnp.sum(lossl) + jnp.sum(spce) + jnp.sum(s_top)) / n_total
